# baseline (speedup 1.0000x reference)
import functools

import jax
import jax.numpy as jnp
from jax import lax
from jax.experimental import pallas as pl
from jax.experimental.pallas import tpu as pltpu
from jax.experimental.pallas import tpu_sc as plsc

_G = 128
_LANES = 16


def _sc_gather_fn(N, C, NC, NS):
    NW = NC * NS
    bpw = N // NW
    nchunk = bpw // _G

    mesh = plsc.VectorSubcoreMesh(core_axis_name="c", subcore_axis_name="s")

    @functools.partial(
        pl.kernel,
        out_type=jax.ShapeDtypeStruct((N // _G, _G), jnp.float32),
        mesh=mesh,
        scratch_types=[
            pltpu.VMEM((bpw,), jnp.int32),
            pltpu.VMEM((nchunk, _G), jnp.int32),
            pltpu.VMEM((nchunk, _G), jnp.float32),
            pltpu.SemaphoreType.DMA,
        ],
    )
    def sc_gather(t_hbm, sim_hbm, simt_out, t_v, idx_v, sim_v, sem):
        wid = lax.axis_index("s") * NC + lax.axis_index("c")
        base = wid * bpw
        pltpu.sync_copy(t_hbm.at[pl.ds(base, bpw)], t_v)
        for j in range(bpw // _LANES):
            tv = t_v[pl.ds(j * _LANES, _LANES)]
            rows = lax.iota(jnp.int32, _LANES) + (base + j * _LANES)
            idx_v[j // (_G // _LANES),
                  pl.ds((j % (_G // _LANES)) * _LANES, _LANES)] = (
                rows * C + tv)
        copies = [pltpu.async_copy(sim_hbm.at[idx_v.at[c]], sim_v.at[c], sem)
                  for c in range(nchunk)]
        for cp in copies:
            cp.wait()
        pltpu.sync_copy(sim_v, simt_out.at[pl.ds(wid * nchunk, nchunk)])

    return sc_gather


def _tc_a_fn(S, C, R):
    def body(x_ref, t_ref, a_ref):
        cols = lax.broadcasted_iota(jnp.int32, (R, _G, C), 2)
        oh = cols == t_ref[...][:, :, None]
        x = x_ref[...]
        m = jnp.max(x, axis=2)
        s = jnp.sum(jnp.exp(x - m[:, :, None]), axis=2)
        lse = m + jnp.log(s)
        logit_t = jnp.sum(jnp.where(oh, x, 0.0), axis=2)
        a_ref[...] = lse - logit_t

    return pl.pallas_call(
        body,
        grid=(S // R,),
        in_specs=[
            pl.BlockSpec((R, _G, C), lambda i: (i, 0, 0)),
            pl.BlockSpec((R, _G), lambda i: (i, 0)),
        ],
        out_specs=pl.BlockSpec((R, _G), lambda i: (i, 0)),
        out_shape=jax.ShapeDtypeStruct((S, _G), jnp.float32),
    )


def _tc_combine_fn(S):
    def body(a_ref, simt_ref, out_ref):
        w = 10.0 / (1.0 + jnp.exp(4.0 * simt_ref[...]))
        out_ref[0, 0] = jnp.sum(w * a_ref[...])

    return pl.pallas_call(
        body,
        out_specs=pl.BlockSpec(memory_space=pltpu.MemorySpace.SMEM),
        out_shape=jax.ShapeDtypeStruct((1, 1), jnp.float32),
    )


def kernel(src_logits, hoi_text_similarity, target_classes_i):
    N, C = src_logits.shape
    S = N // _G
    t = target_classes_i.astype(jnp.int32)

    info = plsc.get_sparse_core_info()
    simt = _sc_gather_fn(N, C, info.num_cores, info.num_subcores)(
        t, hoi_text_similarity.reshape(-1))

    a = _tc_a_fn(S, C, 16)(src_logits.reshape(S, _G, C), t.reshape(S, _G))
    out = _tc_combine_fn(S)(a, simt)
    return out[0, 0] / N


# speedup vs baseline: 1.2175x; 1.2175x over previous
import jax
import jax.numpy as jnp
from jax import lax
from jax.experimental import pallas as pl
from jax.experimental.pallas import tpu as pltpu

_G = 128


def _tc_loss_fn(N, C, R):
    def body(x_ref, s_ref, t_ref, out_ref):
        i = pl.program_id(0)

        @pl.when(i == 0)
        def _init():
            out_ref[0, 0] = 0.0

        x = x_ref[...].reshape(R, _G, C)
        sv = s_ref[...].reshape(R, _G, C)
        cols = lax.broadcasted_iota(jnp.int32, (R, _G, C), 2)
        oh = cols == t_ref[...][:, :, None]
        m = jnp.max(x, axis=2)
        s = jnp.sum(jnp.exp(x - m[:, :, None]), axis=2)
        lse = m + jnp.log(s)
        logit_t = jnp.sum(jnp.where(oh, x, 0.0), axis=2)
        sim_t = jnp.sum(jnp.where(oh, sv, 0.0), axis=2)
        w = 10.0 / (1.0 + jnp.exp(4.0 * sim_t))
        out_ref[0, 0] += jnp.sum(w * (lse - logit_t))

    return pl.pallas_call(
        body,
        grid=(N // (R * _G),),
        in_specs=[
            pl.BlockSpec((R * _G, C), lambda i: (i, 0)),
            pl.BlockSpec((R * _G, C), lambda i: (i, 0)),
            pl.BlockSpec((R, _G), lambda i: (i, 0)),
        ],
        out_specs=pl.BlockSpec(memory_space=pltpu.MemorySpace.SMEM),
        out_shape=jax.ShapeDtypeStruct((1, 1), jnp.float32),
        compiler_params=pltpu.CompilerParams(
            dimension_semantics=("arbitrary",)),
    )


def kernel(src_logits, hoi_text_similarity, target_classes_i):
    N, C = src_logits.shape
    t2 = target_classes_i.astype(jnp.int32).reshape(N // _G, _G)
    R = 16
    out = _tc_loss_fn(N, C, R)(src_logits, hoi_text_similarity, t2)
    return out[0, 0] / N
